# baseline (speedup 1.0000x reference)
import functools

import jax
import jax.numpy as jnp
from jax import lax
from jax.experimental import pallas as pl
from jax.experimental.pallas import tpu as pltpu
from jax.experimental.pallas import tpu_sc as plsc

N = 16384
DIN = 128
SD = 4
PD = 64
K = 16
NSEG = 8
SEG = N // NSEG


_PROJ_R = 2048


_PD_PAD = 128


def _proj_body(x_ref, wsT_ref, wpT_ref, bp_ref, space_ref, spaceT_ref, prop_ref):
    x = x_ref[...]
    sp = jnp.dot(x, wsT_ref[...], preferred_element_type=jnp.float32)
    space_ref[...] = sp
    spaceT_ref[...] = sp.T
    prop_ref[...] = (
        jnp.dot(x, wpT_ref[...], preferred_element_type=jnp.float32) + bp_ref[0:1, :]
    )


def _project(x, wsT, wpT_pad, bp_pad):
    grid = (N // _PROJ_R,)
    return pl.pallas_call(
        _proj_body,
        grid=grid,
        in_specs=[
            pl.BlockSpec((_PROJ_R, DIN), lambda i: (i, 0)),
            pl.BlockSpec((DIN, SD), lambda i: (0, 0)),
            pl.BlockSpec((DIN, _PD_PAD), lambda i: (0, 0)),
            pl.BlockSpec((8, _PD_PAD), lambda i: (0, 0)),
        ],
        out_specs=[
            pl.BlockSpec((_PROJ_R, SD), lambda i: (i, 0)),
            pl.BlockSpec((SD, _PROJ_R), lambda i: (0, i)),
            pl.BlockSpec((_PROJ_R, _PD_PAD), lambda i: (i, 0)),
        ],
        out_shape=[
            jax.ShapeDtypeStruct((N, SD), jnp.float32),
            jax.ShapeDtypeStruct((SD, N), jnp.float32),
            jax.ShapeDtypeStruct((N, _PD_PAD), jnp.float32),
        ],
    )(x, wsT, wpT_pad, bp_pad)


_TOPK_R = 512


_LANE_BITS = 11
_LANE_MASK = (1 << _LANE_BITS) - 1
_EXP_BIAS = 0x08000000


def _topk_body(srow_ref, scolT_ref, idxT_ref, w_ref, *, seg0):
    s = pl.program_id(0) + seg0
    rows = srow_ref[...]
    segT = scolT_ref[...]
    rowsq = jnp.sum(rows * rows, axis=1, keepdims=True)
    colsq = jnp.sum(segT * segT, axis=0, keepdims=True)
    D = rowsq + colsq - 2.0 * jnp.dot(rows, segT, preferred_element_type=jnp.float32)
    D = jnp.clip(D, 0.0, 1e30)
    lane = lax.broadcasted_iota(jnp.int32, (_TOPK_R, SEG), 1)
    kbits = ((lax.bitcast_convert_type(D, jnp.int32) & ~_LANE_MASK) | lane) + _EXP_BIAS
    key = lax.bitcast_convert_type(kbits, jnp.float32)
    big = jnp.float32(jnp.inf)
    ms = []
    m = jnp.min(key, axis=1, keepdims=True)
    ms.append(m)
    for _ in range(K - 1):
        m = jnp.min(jnp.where(key > m, key, big), axis=1, keepdims=True)
        ms.append(m)
    mk = lax.bitcast_convert_type(jnp.concatenate(ms, axis=1), jnp.int32) - _EXP_BIAS
    idxT_ref[...] = ((mk & _LANE_MASK) + s * SEG).T
    d = lax.bitcast_convert_type(mk & ~_LANE_MASK, jnp.float32)
    w_ref[...] = jnp.exp(-10.0 * d)


def _topk(space, spaceT, seg0, nseg):
    npts = nseg * SEG
    bps = SEG // _TOPK_R
    grid = (nseg, bps)
    return pl.pallas_call(
        functools.partial(_topk_body, seg0=seg0),
        grid=grid,
        in_specs=[
            pl.BlockSpec((_TOPK_R, SD), lambda s, b: ((seg0 * SEG) // _TOPK_R + s * bps + b, 0)),
            pl.BlockSpec((SD, SEG), lambda s, b: (0, seg0 + s)),
        ],
        out_specs=[
            pl.BlockSpec((K, _TOPK_R), lambda s, b: (0, s * bps + b)),
            pl.BlockSpec((_TOPK_R, K), lambda s, b: (s * bps + b, 0)),
        ],
        out_shape=[
            jax.ShapeDtypeStruct((K, npts), jnp.int32),
            jax.ShapeDtypeStruct((npts, K), jnp.float32),
        ],
    )(space, spaceT)


_SC_NC = 2
_SC_NS = 16
_SC_NW = _SC_NC * _SC_NS
_SC_CH = 256


def _sc_gather(prop, idxT):
    npts = idxT.shape[1]
    B = K * npts
    b_per_w = B // _SC_NW
    nb = b_per_w // _SC_CH
    mesh = plsc.VectorSubcoreMesh(core_axis_name="c", subcore_axis_name="s")

    @functools.partial(
        pl.kernel,
        mesh=mesh,
        out_type=jax.ShapeDtypeStruct((B, _PD_PAD), jnp.float32),
        scratch_types=[
            pltpu.VMEM((_SC_CH,), jnp.int32),
            pltpu.VMEM((_SC_CH,), jnp.int32),
            pltpu.VMEM((_SC_CH, _PD_PAD), jnp.float32),
            pltpu.VMEM((_SC_CH, _PD_PAD), jnp.float32),
            pltpu.SemaphoreType.DMA,
            pltpu.SemaphoreType.DMA,
            pltpu.SemaphoreType.DMA,
            pltpu.SemaphoreType.DMA,
        ],
    )
    def gather_kernel(prop_hbm, idx_hbm, out_hbm, i0, i1, r0, r1, g0, g1, w0, w1):
        wid = lax.axis_index("s") * _SC_NC + lax.axis_index("c")
        base = wid * b_per_w
        idx_v = (i0, i1)
        rows_v = (r0, r1)
        gsem = (g0, g1)
        wsem = (w0, w1)

        def out_slc(c):
            return out_hbm.at[pl.ds(base + c * _SC_CH, _SC_CH)]

        for c in range(nb):
            b = c % 2
            if c >= 2:
                pltpu.make_async_copy(rows_v[b], out_slc(c - 2), wsem[b]).wait()
            off = base + c * _SC_CH
            pltpu.sync_copy(idx_hbm.at[off // npts, pl.ds(off % npts, _SC_CH)], idx_v[b])
            pltpu.async_copy(prop_hbm.at[idx_v[b]], rows_v[b], gsem[b])
            if c >= 1:
                pltpu.make_async_copy(prop_hbm.at[idx_v[1 - b]], rows_v[1 - b], gsem[1 - b]).wait()
                pltpu.async_copy(rows_v[1 - b], out_slc(c - 1), wsem[1 - b])
        bl = (nb - 1) % 2
        pltpu.make_async_copy(prop_hbm.at[idx_v[bl]], rows_v[bl], gsem[bl]).wait()
        pltpu.async_copy(rows_v[bl], out_slc(nb - 1), wsem[bl])
        pltpu.make_async_copy(rows_v[1 - bl], out_slc(nb - 2), wsem[1 - bl]).wait()
        pltpu.make_async_copy(rows_v[bl], out_slc(nb - 1), wsem[bl]).wait()

    return gather_kernel(prop, idxT)


_POOL_R = 1024


def _pool_body(g_ref, w_ref, o_ref):
    acc = None
    mx = None
    for k in range(K):
        t = g_ref[k][:, :PD] * w_ref[:, k : k + 1]
        acc = t if acc is None else acc + t
        mx = t if mx is None else jnp.maximum(mx, t)
    o_ref[...] = jnp.concatenate([acc * (1.0 / K), mx], axis=1)


def _pool(g, w):
    npts = g.shape[1]
    grid = (npts // _POOL_R,)
    return pl.pallas_call(
        _pool_body,
        grid=grid,
        in_specs=[
            pl.BlockSpec((K, _POOL_R, _PD_PAD), lambda i: (0, i, 0)),
            pl.BlockSpec((_POOL_R, K), lambda i: (i, 0)),
        ],
        out_specs=pl.BlockSpec((_POOL_R, 2 * PD), lambda i: (i, 0)),
        out_shape=jax.ShapeDtypeStruct((npts, 2 * PD), jnp.float32),
    )(g, w)


_CHUNK_SEGS = (3, 3, 1, 1)


def kernel(x, row_splits, W_space, b_space, W_prop, b_prop):
    wpT_pad = jnp.zeros((DIN, _PD_PAD), jnp.float32).at[:, :PD].set(W_prop.T)
    bp_pad = jnp.broadcast_to(jnp.pad(b_prop, (0, _PD_PAD - PD))[None, :], (8, _PD_PAD))
    space, spaceT, prop = _project(x, W_space.T, wpT_pad, bp_pad)
    seg0 = 0
    idxw = []
    for nseg_c in _CHUNK_SEGS:
        idxw.append(_topk(space, spaceT, seg0, nseg_c))
        seg0 += nseg_c
    gs = [
        _sc_gather(prop, idxT).reshape(K, idxT.shape[1], _PD_PAD)
        for idxT, _ in idxw
    ]
    outs = [_pool(g, w) for g, (_, w) in zip(gs, idxw)]
    return jnp.concatenate(outs, axis=0)


# speedup vs baseline: 1.0361x; 1.0361x over previous
import functools

import jax
import jax.numpy as jnp
from jax import lax
from jax.experimental import pallas as pl
from jax.experimental.pallas import tpu as pltpu
from jax.experimental.pallas import tpu_sc as plsc

N = 16384
DIN = 128
SD = 4
PD = 64
K = 16
NSEG = 8
SEG = N // NSEG


_PROJ_R = 2048


_PD_PAD = 128


def _proj_body(x_ref, wsT_ref, wpT_ref, bp_ref, space_ref, spaceT_ref, prop_ref):
    x = x_ref[...]
    sp = jnp.dot(x, wsT_ref[...], preferred_element_type=jnp.float32)
    space_ref[...] = sp
    spaceT_ref[...] = sp.T
    prop_ref[...] = (
        jnp.dot(x, wpT_ref[...], preferred_element_type=jnp.float32) + bp_ref[0:1, :]
    )


def _project(x, wsT, wpT_pad, bp_pad):
    grid = (N // _PROJ_R,)
    return pl.pallas_call(
        _proj_body,
        grid=grid,
        in_specs=[
            pl.BlockSpec((_PROJ_R, DIN), lambda i: (i, 0)),
            pl.BlockSpec((DIN, SD), lambda i: (0, 0)),
            pl.BlockSpec((DIN, _PD_PAD), lambda i: (0, 0)),
            pl.BlockSpec((8, _PD_PAD), lambda i: (0, 0)),
        ],
        out_specs=[
            pl.BlockSpec((_PROJ_R, SD), lambda i: (i, 0)),
            pl.BlockSpec((SD, _PROJ_R), lambda i: (0, i)),
            pl.BlockSpec((_PROJ_R, _PD_PAD), lambda i: (i, 0)),
        ],
        out_shape=[
            jax.ShapeDtypeStruct((N, SD), jnp.float32),
            jax.ShapeDtypeStruct((SD, N), jnp.float32),
            jax.ShapeDtypeStruct((N, _PD_PAD), jnp.float32),
        ],
    )(x, wsT, wpT_pad, bp_pad)


_TOPK_R = 512


_LANE_BITS = 11
_LANE_MASK = (1 << _LANE_BITS) - 1
_EXP_BIAS = 0x08000000


def _topk_body(srow_ref, scolT_ref, idxT_ref, w_ref, *, seg0):
    s = pl.program_id(0) + seg0
    rows = srow_ref[...]
    segT = scolT_ref[...]
    rowsq = jnp.sum(rows * rows, axis=1, keepdims=True)
    colsq = jnp.sum(segT * segT, axis=0, keepdims=True)
    D = rowsq + colsq - 2.0 * jnp.dot(rows, segT, preferred_element_type=jnp.float32)
    D = jnp.clip(D, 0.0, 1e30)
    lane = lax.broadcasted_iota(jnp.int32, (_TOPK_R, SEG), 1)
    kbits = ((lax.bitcast_convert_type(D, jnp.int32) & ~_LANE_MASK) | lane) + _EXP_BIAS
    key = lax.bitcast_convert_type(kbits, jnp.float32)
    big = jnp.float32(jnp.inf)
    ms = []
    m = jnp.min(key, axis=1, keepdims=True)
    ms.append(m)
    for _ in range(K - 1):
        m = jnp.min(jnp.where(key > m, key, big), axis=1, keepdims=True)
        ms.append(m)
    mk = lax.bitcast_convert_type(jnp.concatenate(ms, axis=1), jnp.int32) - _EXP_BIAS
    idxT_ref[...] = ((mk & _LANE_MASK) + s * SEG).T
    d = lax.bitcast_convert_type(mk & ~_LANE_MASK, jnp.float32)
    w_ref[...] = jnp.exp(-10.0 * d)


def _topk(space, spaceT, seg0, nseg):
    npts = nseg * SEG
    bps = SEG // _TOPK_R
    grid = (nseg, bps)
    return pl.pallas_call(
        functools.partial(_topk_body, seg0=seg0),
        grid=grid,
        in_specs=[
            pl.BlockSpec((_TOPK_R, SD), lambda s, b: ((seg0 * SEG) // _TOPK_R + s * bps + b, 0)),
            pl.BlockSpec((SD, SEG), lambda s, b: (0, seg0 + s)),
        ],
        out_specs=[
            pl.BlockSpec((K, _TOPK_R), lambda s, b: (0, s * bps + b)),
            pl.BlockSpec((_TOPK_R, K), lambda s, b: (s * bps + b, 0)),
        ],
        out_shape=[
            jax.ShapeDtypeStruct((K, npts), jnp.int32),
            jax.ShapeDtypeStruct((npts, K), jnp.float32),
        ],
    )(space, spaceT)


_SC_NC = 2
_SC_NS = 16
_SC_NW = _SC_NC * _SC_NS
_SC_CH = 256


def _sc_gather(prop, idxT):
    npts = idxT.shape[1]
    B = K * npts
    b_per_w = B // _SC_NW
    nb = b_per_w // _SC_CH
    mesh = plsc.VectorSubcoreMesh(core_axis_name="c", subcore_axis_name="s")

    @functools.partial(
        pl.kernel,
        mesh=mesh,
        out_type=jax.ShapeDtypeStruct((B, _PD_PAD), jnp.float32),
        scratch_types=[
            pltpu.VMEM((_SC_CH,), jnp.int32),
            pltpu.VMEM((_SC_CH,), jnp.int32),
            pltpu.VMEM((_SC_CH, _PD_PAD), jnp.float32),
            pltpu.VMEM((_SC_CH, _PD_PAD), jnp.float32),
            pltpu.SemaphoreType.DMA,
            pltpu.SemaphoreType.DMA,
            pltpu.SemaphoreType.DMA,
            pltpu.SemaphoreType.DMA,
        ],
    )
    def gather_kernel(prop_hbm, idx_hbm, out_hbm, i0, i1, r0, r1, g0, g1, w0, w1):
        wid = lax.axis_index("s") * _SC_NC + lax.axis_index("c")
        base = wid * b_per_w
        idx_v = (i0, i1)
        rows_v = (r0, r1)
        gsem = (g0, g1)
        wsem = (w0, w1)

        def out_slc(c):
            return out_hbm.at[pl.ds(base + c * _SC_CH, _SC_CH)]

        for c in range(nb):
            b = c % 2
            if c >= 2:
                pltpu.make_async_copy(rows_v[b], out_slc(c - 2), wsem[b]).wait()
            off = base + c * _SC_CH
            pltpu.sync_copy(idx_hbm.at[off // npts, pl.ds(off % npts, _SC_CH)], idx_v[b])
            pltpu.async_copy(prop_hbm.at[idx_v[b]], rows_v[b], gsem[b])
            if c >= 1:
                pltpu.make_async_copy(prop_hbm.at[idx_v[1 - b]], rows_v[1 - b], gsem[1 - b]).wait()
                pltpu.async_copy(rows_v[1 - b], out_slc(c - 1), wsem[1 - b])
        bl = (nb - 1) % 2
        pltpu.make_async_copy(prop_hbm.at[idx_v[bl]], rows_v[bl], gsem[bl]).wait()
        pltpu.async_copy(rows_v[bl], out_slc(nb - 1), wsem[bl])
        pltpu.make_async_copy(rows_v[1 - bl], out_slc(nb - 2), wsem[1 - bl]).wait()
        pltpu.make_async_copy(rows_v[bl], out_slc(nb - 1), wsem[bl]).wait()

    return gather_kernel(prop, idxT)


_POOL_R = 1024


def _pool_body(g_ref, w_ref, o_ref):
    acc = None
    mx = None
    for k in range(K):
        t = g_ref[k][:, :PD] * w_ref[:, k : k + 1]
        acc = t if acc is None else acc + t
        mx = t if mx is None else jnp.maximum(mx, t)
    o_ref[...] = jnp.concatenate([acc * (1.0 / K), mx], axis=1)


def _pool(g, w):
    npts = g.shape[1]
    grid = (npts // _POOL_R,)
    return pl.pallas_call(
        _pool_body,
        grid=grid,
        in_specs=[
            pl.BlockSpec((K, _POOL_R, _PD_PAD), lambda i: (0, i, 0)),
            pl.BlockSpec((_POOL_R, K), lambda i: (i, 0)),
        ],
        out_specs=pl.BlockSpec((_POOL_R, 2 * PD), lambda i: (i, 0)),
        out_shape=jax.ShapeDtypeStruct((npts, 2 * PD), jnp.float32),
    )(g, w)


_CHUNK_SEGS = (2, 2, 2, 2)


def kernel(x, row_splits, W_space, b_space, W_prop, b_prop):
    wpT_pad = jnp.zeros((DIN, _PD_PAD), jnp.float32).at[:, :PD].set(W_prop.T)
    bp_pad = jnp.broadcast_to(jnp.pad(b_prop, (0, _PD_PAD - PD))[None, :], (8, _PD_PAD))
    space, spaceT, prop = _project(x, W_space.T, wpT_pad, bp_pad)
    seg0 = 0
    idxw = []
    for nseg_c in _CHUNK_SEGS:
        idxw.append(_topk(space, spaceT, seg0, nseg_c))
        seg0 += nseg_c
    gs = [
        _sc_gather(prop, idxT).reshape(K, idxT.shape[1], _PD_PAD)
        for idxT, _ in idxw
    ]
    outs = [_pool(g, w) for g, (_, w) in zip(gs, idxw)]
    return jnp.concatenate(outs, axis=0)
